# baseline (speedup 1.0000x reference)
import jax
import jax.numpy as jnp
from jax import lax
from jax.experimental import pallas as pl
from jax.experimental.pallas import tpu as pltpu

NUM_CLASSES = 4
MARGIN = 1.0

B = 2
C = 32
H = W = D = 64
K = NUM_CLASSES
HB = 4


def _tc_body(f_ref, t_ref, sums_ref, counts_ref, loss_ref):
    b = pl.program_id(0)
    n = pl.program_id(1)

    @pl.when(jnp.logical_and(b == 0, n == 0))
    def _init():
        sums_ref[...] = jnp.zeros_like(sums_ref)
        counts_ref[...] = jnp.zeros_like(counts_ref)

    f = f_ref[0]
    t = t_ref[0]
    s_total = jnp.sum(jnp.sum(f, axis=(2, 3)), axis=1, keepdims=True)
    s_rest = jnp.zeros_like(s_total)
    n_rest = jnp.zeros((1, 1), jnp.float32)
    for k in range(1, NUM_CLASSES):
        m32 = (t == k).astype(jnp.float32)
        s_k = jnp.sum(jnp.sum(m32 * f, axis=(2, 3)), axis=1, keepdims=True)
        n_k = jnp.sum(jnp.sum(m32, axis=(2, 3)), axis=1, keepdims=True)
        sums_ref[b, :, k : k + 1] += s_k
        counts_ref[b, :, k : k + 1] += n_k
        s_rest = s_rest + s_k
        n_rest = n_rest + n_k
    sums_ref[b, :, 0:1] += s_total - s_rest
    counts_ref[b, :, 0:1] += jnp.float32(HB * W * D) - n_rest

    @pl.when(jnp.logical_and(b == B - 1, n == pl.num_programs(1) - 1))
    def _finalize():
        sums = sums_ref[...]
        counts = counts_ref[...]
        centers = sums / jnp.maximum(counts, 1.0)
        valid = counts[:, 0, :] > 0
        total = jnp.float32(0.0)
        pairs = jnp.float32(0.0)
        for i in range(NUM_CLASSES):
            for j in range(i + 1, NUM_CLASSES):
                diff = centers[:, :, i] - centers[:, :, j]
                dist = jnp.sqrt(jnp.sum(diff * diff, axis=1))
                hinge = jnp.maximum(MARGIN - dist, 0.0)
                m = jnp.logical_and(valid[:, i], valid[:, j]).astype(jnp.float32)
                total = total + jnp.sum(hinge * m)
                pairs = pairs + jnp.sum(m)
        val = jnp.where(pairs > 0, total / jnp.maximum(pairs, 1.0), 0.0)
        loss_ref[...] = val.reshape(1, 1)


def kernel(features, predictions, targets):
    del predictions
    sums, counts, loss = pl.pallas_call(
        _tc_body,
        grid=(B, H // HB),
        in_specs=[
            pl.BlockSpec((1, C, HB, W, D), lambda b, n: (b, 0, n, 0, 0)),
            pl.BlockSpec((1, 1, HB, W, D), lambda b, n: (b, 0, n, 0, 0)),
        ],
        out_specs=[
            pl.BlockSpec((B, C, K), lambda b, n: (0, 0, 0)),
            pl.BlockSpec((B, 1, K), lambda b, n: (0, 0, 0)),
            pl.BlockSpec((1, 1), lambda b, n: (0, 0)),
        ],
        out_shape=[
            jax.ShapeDtypeStruct((B, C, K), jnp.float32),
            jax.ShapeDtypeStruct((B, 1, K), jnp.float32),
            jax.ShapeDtypeStruct((1, 1), jnp.float32),
        ],
    )(features, targets)
    return loss[0, 0]


# speedup vs baseline: 1.0635x; 1.0635x over previous
import jax
import jax.numpy as jnp
from jax import lax
from jax.experimental import pallas as pl
from jax.experimental.pallas import tpu as pltpu

NUM_CLASSES = 4
MARGIN = 1.0

B = 2
C = 32
H = W = D = 64
K = NUM_CLASSES
HB = 8


def _tc_body(f_ref, t_ref, sums_ref, counts_ref, loss_ref):
    b = pl.program_id(0)
    n = pl.program_id(1)

    @pl.when(jnp.logical_and(b == 0, n == 0))
    def _init():
        sums_ref[...] = jnp.zeros_like(sums_ref)
        counts_ref[...] = jnp.zeros_like(counts_ref)

    f = f_ref[0]
    t = t_ref[0]
    s_total = jnp.sum(jnp.sum(f, axis=(2, 3)), axis=1, keepdims=True)
    s_rest = jnp.zeros_like(s_total)
    n_rest = jnp.zeros((1, 1), jnp.float32)
    for k in range(1, NUM_CLASSES):
        m32 = (t == k).astype(jnp.float32)
        s_k = jnp.sum(jnp.sum(m32 * f, axis=(2, 3)), axis=1, keepdims=True)
        n_k = jnp.sum(jnp.sum(m32, axis=(2, 3)), axis=1, keepdims=True)
        sums_ref[b, :, k : k + 1] += s_k
        counts_ref[b, :, k : k + 1] += n_k
        s_rest = s_rest + s_k
        n_rest = n_rest + n_k
    sums_ref[b, :, 0:1] += s_total - s_rest
    counts_ref[b, :, 0:1] += jnp.float32(HB * W * D) - n_rest

    @pl.when(jnp.logical_and(b == B - 1, n == pl.num_programs(1) - 1))
    def _finalize():
        sums = sums_ref[...]
        counts = counts_ref[...]
        centers = sums / jnp.maximum(counts, 1.0)
        valid = counts[:, 0, :] > 0
        total = jnp.float32(0.0)
        pairs = jnp.float32(0.0)
        for i in range(NUM_CLASSES):
            for j in range(i + 1, NUM_CLASSES):
                diff = centers[:, :, i] - centers[:, :, j]
                dist = jnp.sqrt(jnp.sum(diff * diff, axis=1))
                hinge = jnp.maximum(MARGIN - dist, 0.0)
                m = jnp.logical_and(valid[:, i], valid[:, j]).astype(jnp.float32)
                total = total + jnp.sum(hinge * m)
                pairs = pairs + jnp.sum(m)
        val = jnp.where(pairs > 0, total / jnp.maximum(pairs, 1.0), 0.0)
        loss_ref[...] = val.reshape(1, 1)


def kernel(features, predictions, targets):
    del predictions
    sums, counts, loss = pl.pallas_call(
        _tc_body,
        grid=(B, H // HB),
        in_specs=[
            pl.BlockSpec((1, C, HB, W, D), lambda b, n: (b, 0, n, 0, 0)),
            pl.BlockSpec((1, 1, HB, W, D), lambda b, n: (b, 0, n, 0, 0)),
        ],
        out_specs=[
            pl.BlockSpec((B, C, K), lambda b, n: (0, 0, 0)),
            pl.BlockSpec((B, 1, K), lambda b, n: (0, 0, 0)),
            pl.BlockSpec((1, 1), lambda b, n: (0, 0)),
        ],
        out_shape=[
            jax.ShapeDtypeStruct((B, C, K), jnp.float32),
            jax.ShapeDtypeStruct((B, 1, K), jnp.float32),
            jax.ShapeDtypeStruct((1, 1), jnp.float32),
        ],
    )(features, targets)
    return loss[0, 0]
